# baseline (speedup 1.0000x reference)
import functools

import jax
import jax.numpy as jnp
from jax import lax
from jax.experimental import pallas as pl
from jax.experimental.pallas import tpu as pltpu
from jax.experimental.pallas import tpu_sc as plsc

BATCH = 128
VOCAB = 100000
NC = 2
NS = 16
L = 16
NW = NC * NS
NB = BATCH // L

VSPLIT = 53248
BKV = 512
TC_STEPS = VSPLIT // BKV

SC_RANGE = VOCAB - VSPLIT
VW = 184
NCHK = 8
STRIP = NCHK * VW
STRIDE = (SC_RANGE - STRIP) // (NW - 1) // 8 * 8
NEG_INF = float("-inf")

_mesh = plsc.VectorSubcoreMesh(core_axis_name="c", subcore_axis_name="s")


@functools.partial(
    pl.kernel,
    out_type=(
        jax.ShapeDtypeStruct((NC * BATCH,), jnp.int32),
        jax.ShapeDtypeStruct((NC * BATCH,), jnp.float32),
    ),
    mesh=_mesh,
    scratch_types=[
        pltpu.VMEM((VW, BATCH), jnp.float32),
        pltpu.VMEM((VW, BATCH), jnp.float32),
        pltpu.VMEM((BATCH,), jnp.float32),
        pltpu.VMEM((BATCH,), jnp.int32),
        pltpu.VMEM((NS * L,), jnp.float32),
        pltpu.VMEM((NS * L,), jnp.int32),
        pltpu.VMEM((L,), jnp.float32),
        pltpu.VMEM((L,), jnp.int32),
        pltpu.VMEM_SHARED((NS * BATCH,), jnp.float32),
        pltpu.VMEM_SHARED((NS * BATCH,), jnp.int32),
        pltpu.SemaphoreType.DMA,
        pltpu.SemaphoreType.DMA,
    ],
)
def _argmax_sc(xt_hbm, idx_hbm, val_hbm, buf0, buf1, stv, sti, gv, gi,
               rv, ri, sval, sidx, sem0, sem1):
    cid = lax.axis_index("c")
    sid = lax.axis_index("s")
    wid = sid * NC + cid
    start = pl.multiple_of(
        VSPLIT + lax.min(wid * STRIDE, SC_RANGE - STRIP), 8)

    bufs = (buf0, buf1)
    sems = (sem0, sem1)

    def issue(t):
        return pltpu.async_copy(
            xt_hbm.at[pl.ds(start + t * VW, VW)], bufs[t % 2], sems[t % 2])

    bvs = [jnp.full((L,), NEG_INF, jnp.float32) for _ in range(NB)]
    bps = [jnp.zeros((L,), jnp.int32) for _ in range(NB)]

    handles = [None] * NCHK
    handles[0] = issue(0)
    for t in range(NCHK):
        if t + 1 < NCHK:
            handles[t + 1] = issue(t + 1)
        handles[t].wait()
        buf = bufs[t % 2]
        base = start + t * VW

        def body(i, carry, buf=buf, base=base):
            bvs, bps = carry
            pos = jnp.full((L,), base + i, jnp.int32)
            nbvs, nbps = [], []
            for u in range(NB):
                v = buf[i, pl.ds(u * L, L)]
                m = v > bvs[u]
                nbvs.append(jnp.where(m, v, bvs[u]))
                nbps.append(jnp.where(m, pos, bps[u]))
            return nbvs, nbps

        bvs, bps = plsc.parallel_loop(
            0, VW, unroll=2, carry=(bvs, bps))(body)

    for u in range(NB):
        stv[pl.ds(u * L, L)] = bvs[u]
        sti[pl.ds(u * L, L)] = bps[u]
    pltpu.sync_copy(stv, sval.at[pl.ds(sid * BATCH, BATCH)])
    pltpu.sync_copy(sti, sidx.at[pl.ds(sid * BATCH, BATCH)])
    plsc.subcore_barrier()

    @pl.when(sid < NB)
    def _():
        for j in range(NS):
            pltpu.sync_copy(
                sval.at[pl.ds(j * BATCH + sid * L, L)],
                gv.at[pl.ds(j * L, L)])
            pltpu.sync_copy(
                sidx.at[pl.ds(j * BATCH + sid * L, L)],
                gi.at[pl.ds(j * L, L)])
        av = gv[pl.ds(0, L)]
        ai = gi[pl.ds(0, L)]
        for j in range(1, NS):
            ov = gv[pl.ds(j * L, L)]
            oi = gi[pl.ds(j * L, L)]
            take = (ov > av) | ((ov == av) & (oi < ai))
            av = jnp.where(take, ov, av)
            ai = jnp.where(take, oi, ai)
        rv[...] = av
        ri[...] = ai
        pltpu.sync_copy(ri, idx_hbm.at[pl.ds(cid * BATCH + sid * L, L)])
        pltpu.sync_copy(rv, val_hbm.at[pl.ds(cid * BATCH + sid * L, L)])


def _tc_body(x_ref, val_ref, idx_ref):
    i = pl.program_id(0)
    v = x_ref[...]
    m = jnp.max(v, axis=0)
    rows = lax.broadcasted_iota(jnp.int32, (BKV, BATCH), 0)
    am = jnp.min(jnp.where(v == m[None, :], rows, BKV), axis=0) + i * BKV
    m = m[None, :]
    am = am[None, :]

    @pl.when(i == 0)
    def _():
        val_ref[...] = m
        idx_ref[...] = am

    @pl.when(i > 0)
    def _():
        rv = val_ref[...]
        take = m > rv
        val_ref[...] = jnp.where(take, m, rv)
        idx_ref[...] = jnp.where(take, am, idx_ref[...])


_argmax_tc = pl.pallas_call(
    _tc_body,
    grid=(TC_STEPS,),
    in_specs=[pl.BlockSpec((BKV, BATCH), lambda i: (i, 0))],
    out_specs=(
        pl.BlockSpec((1, BATCH), lambda i: (0, 0)),
        pl.BlockSpec((1, BATCH), lambda i: (0, 0)),
    ),
    out_shape=(
        jax.ShapeDtypeStruct((1, BATCH), jnp.float32),
        jax.ShapeDtypeStruct((1, BATCH), jnp.int32),
    ),
)


def _merge(va, ia, vb, ib):
    take = (vb > va) | ((vb == va) & (ib < ia))
    return jnp.where(take, vb, va), jnp.where(take, ib, ia)


def kernel(logits):
    xt = logits.T
    sc_idx, sc_val = _argmax_sc(xt)
    tc_val, tc_idx = _argmax_tc(xt)
    vi = sc_val.reshape(NC, BATCH)
    ii = sc_idx.reshape(NC, BATCH)
    v, i = _merge(vi[0], ii[0], vi[1], ii[1])
    v, i = _merge(tc_val[0], tc_idx[0], v, i)
    return i
